# Initial kernel scaffold; baseline (speedup 1.0000x reference)
import jax
import jax.numpy as jnp
from jax.experimental import pallas as pl


def kernel(x, edge_index, edge_attr, batch, atom_emb1, atom_emb2, edge_emb1, edge_emb2, W1, b1, W2, b2, bn_scale, bn_bias):
    raise NotImplementedError("write your pallas kernel here")


# speedup vs baseline: 2.3760x; 2.3760x over previous
import functools

import jax
import jax.numpy as jnp
import numpy as np
from jax import lax
from jax.experimental import pallas as pl
from jax.experimental.pallas import tpu as pltpu
from jax.experimental.pallas import tpu_sc as plsc

N_LAYER = 5
EMB = 128
N_GRAPHS = 64

NC = 2
NS = 16
NW = NC * NS
LANES = 16
CAPW = 88


def _make_sc_spmm(n_pad: int, t_rows: int):
    rpt = n_pad // NW
    scrows = n_pad // NC
    mesh = plsc.VectorSubcoreMesh(core_axis_name="c", subcore_axis_name="s")

    @functools.partial(
        pl.kernel,
        out_type=jax.ShapeDtypeStruct((n_pad, EMB), jnp.float32),
        mesh=mesh,
        scratch_types=[
            pltpu.VMEM_SHARED((scrows, EMB), jnp.float32),
            pltpu.VMEM((CAPW, 128), jnp.int32),
            pltpu.VMEM((CAPW, 128), jnp.int32),
            pltpu.VMEM((128, EMB), jnp.float32),
            pltpu.VMEM((32, EMB), jnp.float32),
            pltpu.SemaphoreType.DMA,
        ],
    )
    def spmm(tbl, pg, pd, out, acc, idxg, idxd, rows, zbuf, sem):
        c = lax.axis_index("c")
        s = lax.axis_index("s")
        wid = c * NS + s

        zv = jnp.zeros((LANES,), jnp.float32)
        for i in range(32):
            for j in range(EMB // LANES):
                zbuf[i, pl.ds(j * LANES, LANES)] = zv
        for k in range(rpt // 32):
            pltpu.sync_copy(zbuf, acc.at[pl.ds(s * rpt + k * 32, 32)])

        pltpu.sync_copy(pg.at[wid], idxg)
        pltpu.sync_copy(pd.at[wid], idxd)

        def step(j, carry):
            pltpu.async_copy(tbl.at[idxg.at[j]], rows, sem).wait()
            pltpu.sync_copy(rows, acc.at[idxd.at[j]], add=True)
            return carry

        lax.fori_loop(0, CAPW, step, 0)

        for k in range(rpt // 64):
            st = s * rpt + k * 64
            pltpu.sync_copy(acc.at[pl.ds(st, 64)], rows.at[pl.ds(0, 64)])
            pltpu.sync_copy(rows.at[pl.ds(0, 64)],
                            out.at[pl.ds(c * scrows + st, 64)])

    return spmm


def _rowsum8(p):
    a = p[0:1]
    for i in range(1, 8):
        a = a + p[i:i + 1]
    return a


def _embed_body(x0, x1, a1, a2, out):
    bn = x0.shape[0]
    oh1 = (x0[:, 0:1] == lax.broadcasted_iota(jnp.int32, (bn, 128), 1))
    oh2 = (x1[:, 0:1] == lax.broadcasted_iota(jnp.int32, (bn, 8), 1))
    hi = lax.Precision.HIGHEST
    out[...] = (
        jnp.dot(oh1.astype(jnp.float32), a1[...],
                preferred_element_type=jnp.float32, precision=hi)
        + jnp.dot(oh2.astype(jnp.float32), a2[...],
                  preferred_element_type=jnp.float32, precision=hi)
    )


def _tbuild_body(hb, tblb, t_out):
    cc = pl.program_id(1)
    t_out[...] = jnp.where(cc < 10, hb[...] + tblb[0, 0:1], 0.0)


def _layer_a_body(aggb, w1b, b1b, w2b, b2b, hpre, ssum):
    i = pl.program_id(0)
    bf = jnp.bfloat16
    hmid = jnp.maximum(
        jnp.dot(aggb[...].astype(bf), w1b[...].astype(bf),
                preferred_element_type=jnp.float32) + b1b[0:1], 0.0)
    hp = jnp.dot(hmid.astype(bf), w2b[...].astype(bf),
                 preferred_element_type=jnp.float32) + b2b[0:1]
    hpre[...] = hp
    ps = hp[0:8]
    for k in range(1, hp.shape[0] // 8):
        ps = ps + hp[k * 8:k * 8 + 8]

    @pl.when(i == 0)
    def _():
        ssum[...] = ps

    @pl.when(i > 0)
    def _():
        ssum[...] += ps


def _make_layer_b_body(inv_n):
    def body(hpb, ssumb, ssq):
        i = pl.program_id(0)
        mean = _rowsum8(ssumb[...]) * inv_n
        d = hpb[...] - mean
        sq = d * d
        ps = sq[0:8]
        for k in range(1, sq.shape[0] // 8):
            ps = ps + sq[k * 8:k * 8 + 8]

        @pl.when(i == 0)
        def _():
            ssq[...] = ps

        @pl.when(i > 0)
        def _():
            ssq[...] += ps
    return body


def _make_norm_t_body(inv_n):
    def body(hpb, ssumb, ssqb, scb, bib, tblb, t_out):
        cc = pl.program_id(1)
        mean = _rowsum8(ssumb[...]) * inv_n
        var = _rowsum8(ssqb[...]) * inv_n
        denom = jnp.sqrt(var + jnp.float32(1e-5))
        h = jnp.maximum((hpb[...] - mean) / denom * scb[0:1] + bib[0:1], 0.0)
        t_out[...] = jnp.where(cc < 10, h + tblb[0, 0:1], 0.0)
    return body


def _make_pool_body(inv_n, nblocks):
    def body(hpb, batchb, ssumb, ssqb, scb, bib, g, accs, accc):
        i = pl.program_id(0)
        bn = hpb.shape[0]
        hi = lax.Precision.HIGHEST
        oh = (batchb[:, 0:1] == lax.broadcasted_iota(jnp.int32, (bn, N_GRAPHS), 1)
              ).astype(jnp.float32)
        dn = (((0,), (0,)), ((), ()))
        ps = lax.dot_general(oh, hpb[...], dn,
                             preferred_element_type=jnp.float32, precision=hi)
        pc = lax.dot_general(oh, jnp.ones((bn, EMB), jnp.float32), dn,
                             preferred_element_type=jnp.float32, precision=hi)

        @pl.when(i == 0)
        def _():
            accs[...] = ps
            accc[...] = pc

        @pl.when(i > 0)
        def _():
            accs[...] += ps
            accc[...] += pc

        @pl.when(i == nblocks - 1)
        def _():
            mean = _rowsum8(ssumb[...]) * inv_n
            var = _rowsum8(ssqb[...]) * inv_n
            denom = jnp.sqrt(var + jnp.float32(1e-5))
            cnt = jnp.maximum(accc[...], 1.0)
            gm = (accs[...] / cnt - mean) / denom * scb[0:1] + bib[0:1]
            g[...] = jnp.where(accc[...] > 0.0, gm, 0.0)
    return body


def kernel(x, edge_index, edge_attr, batch, atom_emb1, atom_emb2, edge_emb1,
           edge_emb2, W1, b1, W2, b2, bn_scale, bn_bias):
    N = x.shape[0]
    E = edge_index.shape[1]
    f32 = jnp.float32
    i32 = jnp.int32
    etot = E + N
    cap = CAPW * 128

    rpt = max(64, -(-(-(-N // NW)) // 64) * 64)
    n_pad = NW * rpt
    scrows = n_pad // NC
    bn = 400
    assert N % bn == 0
    nblocks = N // bn
    inv_n = np.float32(1.0 / N)

    loops = jnp.arange(N, dtype=i32)
    src_full = jnp.concatenate([edge_index[0].astype(i32), loops])
    dst_full = jnp.concatenate([edge_index[1].astype(i32), loops])
    combo_full = jnp.concatenate([
        (edge_attr[:, 0] * 3 + edge_attr[:, 1]).astype(i32),
        jnp.full((N,), 9, i32)])
    perm = jnp.argsort(dst_full, stable=True)
    sdst = dst_full[perm]
    gidx_s = combo_full[perm] * N + src_full[perm]
    tile_e = sdst // rpt
    off = jnp.searchsorted(sdst, jnp.arange(NW + 1, dtype=i32) * rpt,
                           side="left").astype(i32)
    pos = tile_e * cap + (jnp.arange(etot, dtype=i32) - off[tile_e])
    slot = jnp.arange(NW * cap, dtype=i32)
    fill_g = 10 * N + (slot % N)
    fill_d = ((slot // cap) % NS) * rpt + (slot % rpt)
    sdst_local = sdst - (tile_e // NS) * scrows
    pgidx = fill_g.at[pos].set(gidx_s).reshape(NW, CAPW, 128)
    pdst = fill_d.at[pos].set(sdst_local).reshape(NW, CAPW, 128)

    a0 = jnp.array([0, 0, 0, 1, 1, 1, 2, 2, 2, 4], i32)
    a1i = jnp.array([0, 1, 2, 0, 1, 2, 0, 1, 2, 0], i32)
    combo_rows = edge_emb1[:, a0, :] + edge_emb2[:, a1i, :]
    tbl_all = jnp.zeros((N_LAYER, 11, 8, EMB), f32).at[:, :10].set(
        jnp.broadcast_to(combo_rows[:, :, None, :], (N_LAYER, 10, 8, EMB)))
    b1b = jnp.broadcast_to(b1[:, None, :], (N_LAYER, 8, 2 * EMB))
    b2b = jnp.broadcast_to(b2[:, None, :], (N_LAYER, 8, EMB))
    scb = jnp.broadcast_to(bn_scale[:, None, :], (N_LAYER, 8, EMB))
    bib = jnp.broadcast_to(bn_bias[:, None, :], (N_LAYER, 8, EMB))
    a1p = jnp.zeros((128, EMB), f32).at[:atom_emb1.shape[0]].set(atom_emb1)
    a2p = jnp.zeros((8, EMB), f32).at[:atom_emb2.shape[0]].set(atom_emb2)
    x0b = jnp.broadcast_to(x[:, 0:1].astype(i32), (N, 8))
    x1b = jnp.broadcast_to(x[:, 1:2].astype(i32), (N, 8))
    batchb = jnp.broadcast_to(batch[:, None].astype(i32), (N, 8))

    spmm = _make_sc_spmm(n_pad, 11 * N)

    h0 = pl.pallas_call(
        _embed_body,
        grid=(nblocks,),
        in_specs=[
            pl.BlockSpec((bn, 8), lambda i: (i, 0)),
            pl.BlockSpec((bn, 8), lambda i: (i, 0)),
            pl.BlockSpec((128, EMB), lambda i: (0, 0)),
            pl.BlockSpec((8, EMB), lambda i: (0, 0)),
        ],
        out_specs=pl.BlockSpec((bn, EMB), lambda i: (i, 0)),
        out_shape=jax.ShapeDtypeStruct((N, EMB), f32),
    )(x0b, x1b, a1p, a2p)

    tbuild = pl.pallas_call(
        _tbuild_body,
        grid=(nblocks, 11),
        in_specs=[
            pl.BlockSpec((bn, EMB), lambda i, c: (i, 0)),
            pl.BlockSpec((1, 8, EMB), lambda i, c: (c, 0, 0)),
        ],
        out_specs=pl.BlockSpec((bn, EMB), lambda i, c: (c * nblocks + i, 0)),
        out_shape=jax.ShapeDtypeStruct((11 * N, EMB), f32),
    )

    layer_a = pl.pallas_call(
        _layer_a_body,
        grid=(nblocks,),
        in_specs=[
            pl.BlockSpec((bn, EMB), lambda i: (i, 0)),
            pl.BlockSpec((EMB, 2 * EMB), lambda i: (0, 0)),
            pl.BlockSpec((8, 2 * EMB), lambda i: (0, 0)),
            pl.BlockSpec((2 * EMB, EMB), lambda i: (0, 0)),
            pl.BlockSpec((8, EMB), lambda i: (0, 0)),
        ],
        out_specs=[
            pl.BlockSpec((bn, EMB), lambda i: (i, 0)),
            pl.BlockSpec((8, EMB), lambda i: (0, 0)),
        ],
        out_shape=[
            jax.ShapeDtypeStruct((N, EMB), f32),
            jax.ShapeDtypeStruct((8, EMB), f32),
        ],
    )

    layer_b = pl.pallas_call(
        _make_layer_b_body(inv_n),
        grid=(nblocks,),
        in_specs=[
            pl.BlockSpec((bn, EMB), lambda i: (i, 0)),
            pl.BlockSpec((8, EMB), lambda i: (0, 0)),
        ],
        out_specs=pl.BlockSpec((8, EMB), lambda i: (0, 0)),
        out_shape=jax.ShapeDtypeStruct((8, EMB), f32),
    )

    norm_t = pl.pallas_call(
        _make_norm_t_body(inv_n),
        grid=(nblocks, 11),
        in_specs=[
            pl.BlockSpec((bn, EMB), lambda i, c: (i, 0)),
            pl.BlockSpec((8, EMB), lambda i, c: (0, 0)),
            pl.BlockSpec((8, EMB), lambda i, c: (0, 0)),
            pl.BlockSpec((8, EMB), lambda i, c: (0, 0)),
            pl.BlockSpec((8, EMB), lambda i, c: (0, 0)),
            pl.BlockSpec((1, 8, EMB), lambda i, c: (c, 0, 0)),
        ],
        out_specs=pl.BlockSpec((bn, EMB), lambda i, c: (c * nblocks + i, 0)),
        out_shape=jax.ShapeDtypeStruct((11 * N, EMB), f32),
    )

    pool = pl.pallas_call(
        _make_pool_body(inv_n, nblocks),
        grid=(nblocks,),
        in_specs=[
            pl.BlockSpec((bn, EMB), lambda i: (i, 0)),
            pl.BlockSpec((bn, 8), lambda i: (i, 0)),
            pl.BlockSpec((8, EMB), lambda i: (0, 0)),
            pl.BlockSpec((8, EMB), lambda i: (0, 0)),
            pl.BlockSpec((8, EMB), lambda i: (0, 0)),
            pl.BlockSpec((8, EMB), lambda i: (0, 0)),
        ],
        out_specs=pl.BlockSpec((N_GRAPHS, EMB), lambda i: (0, 0)),
        out_shape=jax.ShapeDtypeStruct((N_GRAPHS, EMB), f32),
        scratch_shapes=[
            pltpu.VMEM((N_GRAPHS, EMB), f32),
            pltpu.VMEM((N_GRAPHS, EMB), f32),
        ],
    )

    t_tbl = tbuild(h0, tbl_all[0])
    for l in range(N_LAYER):
        agg = spmm(t_tbl, pgidx, pdst)
        hpre, ssum = layer_a(agg, W1[l], b1b[l], W2[l], b2b[l])
        ssq = layer_b(hpre, ssum)
        if l != N_LAYER - 1:
            t_tbl = norm_t(hpre, ssum, ssq, scb[l], bib[l], tbl_all[l + 1])
        else:
            g = pool(hpre, batchb, ssum, ssq, scb[l], bib[l])
    return g
